# baseline (speedup 1.0000x reference)
import functools
import math

import numpy as np

import jax
import jax.numpy as jnp
from jax import lax
from jax.experimental import pallas as pl
from jax.experimental.pallas import tpu as pltpu
from jax.experimental.pallas import tpu_sc as plsc

F32 = jnp.float32
I32 = jnp.int32

_W = 32
_NW = 32
_NSUB = 16
_CB = 128


def _proj_body(x_ref, w_ref, e_ref, y_ref, z_ref, t_ref, acc_ref, *, q):
    k = pl.program_id(0)
    part = jnp.dot(x_ref[...], w_ref[...], preferred_element_type=F32)

    @pl.when(k == 0)
    def _():
        acc_ref[...] = part
        ei = e_ref[...]
        j = ((ei >= q).astype(I32) + (ei >= 2 * q).astype(I32)
             + (ei >= 3 * q).astype(I32))
        t_ref[...] = 4 * ei - (4 * q - 1) * j

    @pl.when(k == 1)
    def _():
        r = acc_ref[...] + part
        parts_y = []
        parts_z = []
        for jj in range(4):
            blk = r[jj * q:(jj + 1) * q, :]
            parts_y.append(blk[:, :_W])
            parts_z.append(blk[:, _W:])
        y = jnp.concatenate(parts_y, axis=1)
        z = jnp.concatenate(parts_z, axis=1)
        col = lax.broadcasted_iota(I32, (q, 4 * _W), 1)
        y_ref[...] = jnp.where(col % _W == _W - 1, 1.0, y)
        z_ref[...] = z


def _tc_proj(x, w, e2d):
    n, f_in = x.shape
    q = n // 4
    return pl.pallas_call(
        functools.partial(_proj_body, q=q),
        grid=(2,),
        in_specs=[
            pl.BlockSpec((n, f_in // 2), lambda k: (0, k)),
            pl.BlockSpec((f_in // 2, 2 * _W), lambda k: (k, 0)),
            pl.BlockSpec(e2d.shape, lambda k: (0, 0)),
        ],
        out_specs=[
            pl.BlockSpec((q, 128), lambda k: (0, 0)),
            pl.BlockSpec((q, 128), lambda k: (0, 0)),
            pl.BlockSpec(e2d.shape, lambda k: (0, 0)),
        ],
        out_shape=[jax.ShapeDtypeStruct((q, 128), F32),
                   jax.ShapeDtypeStruct((q, 128), F32),
                   jax.ShapeDtypeStruct(e2d.shape, I32)],
        scratch_shapes=[pltpu.VMEM((n, 2 * _W), F32)],
    )(x, w, e2d)


def _lane_tile(row30):
    row32 = jnp.concatenate([row30, jnp.zeros((1, 2), F32)], axis=1)
    return jnp.concatenate([row32] * 4, axis=1)


def _combine_common(s0_ref, s1_ref, z_ref, p_ref, fold_ref, degb_ref,
                    q, layer, relu):
    b_t = _lane_tile(p_ref[pl.ds(3 * layer, 1), :])
    g_t = _lane_tile(p_ref[pl.ds(3 * layer + 1, 1), :])
    be_t = _lane_tile(p_ref[pl.ds(3 * layer + 2, 1), :])
    s = s0_ref[pl.ds(0, q), :] + s1_ref[pl.ds(0, q), :]
    degb = jnp.dot(s, degb_ref[...], preferred_element_type=F32)
    h = s / jnp.maximum(degb, 1.0) + b_t + z_ref[...]
    if relu:
        h = jnp.maximum(h, 0.0)
    mu = jnp.dot(jnp.mean(h, axis=0, keepdims=True), fold_ref[...],
                 preferred_element_type=F32) * 0.25
    xc = h - mu
    var = jnp.dot(jnp.mean(xc * xc, axis=0, keepdims=True), fold_ref[...],
                  preferred_element_type=F32) * 0.25
    return g_t * xc * lax.rsqrt(var + 1e-5) + be_t


def _combine_body(s0_ref, s1_ref, z_ref, p_ref, fold_ref, degb_ref,
                  w_ref, wz_ref, y_ref, zo_ref, *, q, layer, relu):
    hb = _combine_common(s0_ref, s1_ref, z_ref, p_ref, fold_ref, degb_ref,
                         q, layer, relu)
    y = jnp.dot(hb, w_ref[...], preferred_element_type=F32)
    col = lax.broadcasted_iota(I32, (q, 4 * _W), 1)
    y_ref[...] = jnp.where(col % _W == _W - 1, 1.0, y)
    zo_ref[...] = jnp.dot(hb, wz_ref[...], preferred_element_type=F32)


def _tc_combine(s0, s1, z, p, fold, degb, w, wz, layer, relu):
    q = z.shape[0]
    return pl.pallas_call(
        functools.partial(_combine_body, q=q, layer=layer, relu=relu),
        out_shape=[jax.ShapeDtypeStruct((q, 128), F32),
                   jax.ShapeDtypeStruct((q, 128), F32)],
    )(s0, s1, z, p, fold, degb, w, wz)


def _final_body(s0_ref, s1_ref, z_ref, p_ref, fold_ref, degb_ref,
                o_ref, *, q, q2):
    hb = _combine_common(s0_ref, s1_ref, z_ref, p_ref, fold_ref, degb_ref,
                         q, layer=2, relu=False)
    o_ref[...] = jnp.concatenate(
        [hb, jnp.full((q2 - q, 128), -1e30, F32)], axis=0)


def _tc_final(s0, s1, z, p, fold, degb, q2):
    q = z.shape[0]
    return pl.pallas_call(
        functools.partial(_final_body, q=q, q2=q2),
        out_shape=jax.ShapeDtypeStruct((q2, 128), F32),
    )(s0, s1, z, p, fold, degb)


def _head_body(t_ref, w1_ref, b1_ref, w2_ref, b2_ref, o_ref, *, h):
    t = t_ref[...].reshape(_NW, 32, 128)
    p = jnp.max(t, axis=0)
    p = jnp.where(p < -1e29, 0.0, p)
    p = p.reshape(128, _W)[:, :h]
    zz = jnp.dot(p, w1_ref[...], preferred_element_type=F32) \
        + b1_ref[...][None, :]
    zz = jnp.maximum(zz, 0.0)
    o = jnp.dot(zz, w2_ref[...], preferred_element_type=F32) \
        + b2_ref[...][None, :]
    o_ref[...] = 1.0 / (1.0 + jnp.exp(-2.0 * o))


def _tc_head(t, w1, b1, w2, b2):
    return pl.pallas_call(
        functools.partial(_head_body, h=w1.shape[0]),
        out_shape=jax.ShapeDtypeStruct((128, 1), F32),
    )(t, w1, b1, w2, b2)


def _make_segsum(n_nodes, n_ch):
    npad = ((n_nodes + 1 + _NSUB * 8 - 1) // (_NSUB * 8)) * (_NSUB * 8)
    rpt = npad // _NSUB
    mesh = plsc.VectorSubcoreMesh(core_axis_name="c", subcore_axis_name="s")
    K = 5
    ng = n_ch // K
    assert n_ch % (2 * K) == 0 and ng >= 4

    @functools.partial(
        pl.kernel, mesh=mesh,
        compiler_params=pltpu.CompilerParams(use_tc_tiling_on_sc=False),
        out_type=[jax.ShapeDtypeStruct((npad, _W), F32),
                  jax.ShapeDtypeStruct((npad, _W), F32)],
        scratch_types=[
            pltpu.VMEM_SHARED((npad, _W), F32),
            pltpu.VMEM((n_ch, _CB), I32),
            pltpu.VMEM((n_ch, _CB), I32),
            pltpu.VMEM((K, _CB, _W), F32),
            pltpu.VMEM((K, _CB, _W), F32),
            pltpu.SemaphoreType.DMA,
            pltpu.SemaphoreType.DMA,
            pltpu.SemaphoreType.DMA,
            pltpu.SemaphoreType.DMA,
        ],
    )
    def segsum(y_hbm, src_hbm, dst_hbm, zeros_hbm, out0, out1,
               acc, src_v, dst_v, rows0, rows1,
               sem_g0, sem_g1, sem_s0, sem_s1):
        c = lax.axis_index("c")
        s = lax.axis_index("s")
        wid = c * _NSUB + s
        rs = pl.ds(s * rpt, rpt)
        rows = (rows0, rows1)
        semg = (sem_g0, sem_g1)
        sems = (sem_s0, sem_s1)
        pltpu.sync_copy(src_hbm.at[wid], src_v)

        def fire_g(g, b):
            for k in range(K):
                pltpu.async_copy(y_hbm.at[src_v.at[g * K + k]],
                                 rows[b].at[k], semg[b])

        def drain_g(g, b):
            for k in range(K):
                pltpu.make_async_copy(y_hbm.at[src_v.at[g * K + k]],
                                      rows[b].at[k], semg[b]).wait()

        def fire_s(g, b):
            for k in range(K):
                pltpu.async_copy(rows[b].at[k],
                                 acc.at[dst_v.at[g * K + k]], sems[b],
                                 add=True)

        def drain_s(g, b):
            for k in range(K):
                pltpu.make_async_copy(rows[b].at[k],
                                      acc.at[dst_v.at[g * K + k]],
                                      sems[b]).wait()

        fire_g(0, 0)
        pltpu.sync_copy(dst_hbm.at[wid], dst_v)
        pltpu.sync_copy(zeros_hbm, acc.at[rs])
        plsc.subcore_barrier()
        drain_g(0, 0)
        fire_s(0, 0)
        fire_g(1, 1)

        def pair(t, carry):
            g1 = 2 * t + 1
            drain_g(g1, 1)
            fire_s(g1, 1)
            drain_s(g1 - 1, 0)
            fire_g(g1 + 1, 0)
            g2 = 2 * t + 2
            drain_g(g2, 0)
            fire_s(g2, 0)
            drain_s(g2 - 1, 1)
            fire_g(g2 + 1, 1)
            return carry
        lax.fori_loop(0, (ng - 2) // 2, pair, 0)

        g_last = ng - 1
        drain_g(g_last, 1)
        fire_s(g_last, 1)
        drain_s(g_last - 1, 0)
        drain_s(g_last, 1)

        plsc.subcore_barrier()

        @pl.when(c == 0)
        def _():
            pltpu.sync_copy(acc.at[rs], out0.at[rs])

        @pl.when(c == 1)
        def _():
            pltpu.sync_copy(acc.at[rs], out1.at[rs])

    return segsum, npad


def _make_segmax(n_per_tile, n_graphs):
    mesh = plsc.VectorSubcoreMesh(core_axis_name="c", subcore_axis_name="s")

    @functools.partial(
        pl.kernel, mesh=mesh,
        compiler_params=pltpu.CompilerParams(use_tc_tiling_on_sc=False),
        out_type=jax.ShapeDtypeStruct((_NW, n_graphs, _W), F32),
        scratch_types=[
            pltpu.VMEM((n_per_tile, _W), F32),
            pltpu.VMEM((n_per_tile,), I32),
            pltpu.VMEM((n_graphs, _W), F32),
        ],
    )
    def segmax(h_hbm, batch_hbm, neg_hbm, out, h_v, b_v, acc_v):
        c = lax.axis_index("c")
        s = lax.axis_index("s")
        wid = c * _NSUB + s
        n0 = wid * n_per_tile
        pltpu.sync_copy(h_hbm.at[pl.ds(n0, n_per_tile)], h_v)
        pltpu.sync_copy(batch_hbm.at[pl.ds(n0, n_per_tile)], b_v)
        pltpu.sync_copy(neg_hbm, acc_v)

        def body(j, carry):
            n0 = j * 16
            gvec = b_v[pl.ds(n0, 16)]
            for k in range(16):
                g = gvec[k]
                nk = n0 + k
                lo = acc_v[g, pl.ds(0, 16)]
                hi = acc_v[g, pl.ds(16, 16)]
                acc_v[g, pl.ds(0, 16)] = jnp.maximum(lo, h_v[nk, pl.ds(0, 16)])
                acc_v[g, pl.ds(16, 16)] = jnp.maximum(hi, h_v[nk, pl.ds(16, 16)])
            return carry
        lax.fori_loop(0, n_per_tile // 16, body, 0)

        pltpu.sync_copy(acc_v, out.at[wid])

    return segmax


def kernel(x, edge_index, edge_attr, batch,
           Wl1, Wr1, b1, Wl2, Wr2, b2, Wl3, Wr3, b3,
           g1, be1, g2, be2, g3, be3, lin1_W, lin1_b, lin2_W, lin2_b):
    n, f_in = x.shape
    e = edge_index.shape[1]
    h = Wl1.shape[1]
    g_graphs = 128

    q = n // 4

    n_ch = math.ceil(e / (_NW * _CB))
    e_pad = _NW * n_ch * _CB
    segsum, npad = _make_segsum(n, n_ch)

    pad_i = np.arange(e_pad - e)
    src_pad = jnp.asarray(4 * (pad_i % q), dtype=I32)
    dst_pad = jnp.asarray(n + pad_i % (npad - n), dtype=I32)
    e2d = edge_index.reshape(2 * e // 128, 128)
    zeros_t = jnp.asarray(np.zeros((npad // _NSUB, _W), np.float32))

    npt = math.ceil(n / (_NW * 8)) * 8
    n2 = _NW * npt
    batch_packed = batch.reshape(4, q).T.reshape(n)
    batch_p = jnp.concatenate([batch_packed, jnp.zeros((n2 - n,), I32)])
    neg = jnp.asarray(np.full((g_graphs, _W), -1e30, np.float32))
    segmax = _make_segmax(npt, g_graphs)

    def pad32(wl):
        w = jnp.zeros((_W, _W), F32)
        return w.at[:h, :h].set(wl)

    def blockdiag4(w32):
        z = jnp.zeros((_W, _W), F32)
        return jnp.block([[w32 if i == j else z for j in range(4)]
                          for i in range(4)])

    w1c = jnp.zeros((f_in, 2 * _W), F32)
    w1c = w1c.at[:, :h].set(Wl1).at[:, _W:_W + h].set(Wr1)
    w2y, w2z = blockdiag4(pad32(Wl2)), blockdiag4(pad32(Wr2))
    w3y, w3z = blockdiag4(pad32(Wl3)), blockdiag4(pad32(Wr3))

    bn_p = jnp.stack([b1, g1, be1, b2, g2, be2, b3, g3, be3])

    lane = np.arange(128)
    fold = jnp.asarray(
        (lane[:, None] % _W == lane[None, :] % _W).astype(np.float32))
    degb = jnp.asarray(
        (lane[:, None] == _W * (lane[None, :] // _W) + _W - 1)
        .astype(np.float32))

    def sc_view(ypk):
        return ypk.reshape(n, _W)

    def tc_view(part):
        return part.reshape(npad // 4, 128)

    y1, z1, t2d = _tc_proj(x, w1c, e2d)
    t_flat = t2d.reshape(2 * e)
    src = jnp.concatenate([t_flat[:e], src_pad])
    dst = jnp.concatenate([t_flat[e:], dst_pad])
    src_r = src.reshape(_NW, n_ch, _CB)
    dst_r = dst.reshape(_NW, n_ch, _CB)
    p0, p1 = segsum(sc_view(y1), src_r, dst_r, zeros_t)
    y2, z2 = _tc_combine(tc_view(p0), tc_view(p1), z1, bn_p,
                         fold, degb, w2y, w2z, layer=0, relu=True)
    p0, p1 = segsum(sc_view(y2), src_r, dst_r, zeros_t)
    y3, z3 = _tc_combine(tc_view(p0), tc_view(p1), z2, bn_p,
                         fold, degb, w3y, w3z, layer=1, relu=True)
    p0, p1 = segsum(sc_view(y3), src_r, dst_r, zeros_t)
    h3 = _tc_final(tc_view(p0), tc_view(p1), z3, bn_p,
                   fold, degb, n2 // 4)
    t = segmax(h3.reshape(n2, _W), batch_p, neg)
    return _tc_head(t.reshape(_NW * g_graphs // 4, 128),
                    lin1_W, lin1_b, lin2_W, lin2_b)


# speedup vs baseline: 1.0070x; 1.0070x over previous
import functools
import math

import numpy as np

import jax
import jax.numpy as jnp
from jax import lax
from jax.experimental import pallas as pl
from jax.experimental.pallas import tpu as pltpu
from jax.experimental.pallas import tpu_sc as plsc

F32 = jnp.float32
I32 = jnp.int32

_W = 32
_NW = 32
_NSUB = 16
_CB = 128


def _proj_body(x_ref, w_ref, e_ref, y_ref, z_ref, t_ref, *, q):
    ei = e_ref[...]
    j = ((ei >= q).astype(I32) + (ei >= 2 * q).astype(I32)
         + (ei >= 3 * q).astype(I32))
    t_ref[...] = 4 * ei - (4 * q - 1) * j
    parts_y = []
    parts_z = []
    for jj in range(4):
        r = jnp.dot(x_ref[pl.ds(jj * q, q), :], w_ref[...],
                    preferred_element_type=F32)
        parts_y.append(r[:, :_W])
        parts_z.append(r[:, _W:])
    y = jnp.concatenate(parts_y, axis=1)
    z = jnp.concatenate(parts_z, axis=1)
    col = lax.broadcasted_iota(I32, (q, 4 * _W), 1)
    y_ref[...] = jnp.where(col % _W == _W - 1, 1.0, y)
    z_ref[...] = z


def _tc_proj(x, w, e2d):
    q = x.shape[0] // 4
    return pl.pallas_call(
        functools.partial(_proj_body, q=q),
        out_shape=[jax.ShapeDtypeStruct((q, 128), F32),
                   jax.ShapeDtypeStruct((q, 128), F32),
                   jax.ShapeDtypeStruct(e2d.shape, I32)],
    )(x, w, e2d)


def _lane_tile(row30):
    row32 = jnp.concatenate([row30, jnp.zeros((1, 2), F32)], axis=1)
    return jnp.concatenate([row32] * 4, axis=1)


def _combine_common(s0_ref, s1_ref, z_ref, p_ref, fold_ref, degb_ref,
                    q, layer, relu):
    b_t = _lane_tile(p_ref[pl.ds(3 * layer, 1), :])
    g_t = _lane_tile(p_ref[pl.ds(3 * layer + 1, 1), :])
    be_t = _lane_tile(p_ref[pl.ds(3 * layer + 2, 1), :])
    s = s0_ref[pl.ds(0, q), :] + s1_ref[pl.ds(0, q), :]
    degb = jnp.dot(s, degb_ref[...], preferred_element_type=F32)
    h = s / jnp.maximum(degb, 1.0) + b_t + z_ref[...]
    if relu:
        h = jnp.maximum(h, 0.0)
    mu = jnp.dot(jnp.mean(h, axis=0, keepdims=True), fold_ref[...],
                 preferred_element_type=F32) * 0.25
    xc = h - mu
    var = jnp.dot(jnp.mean(xc * xc, axis=0, keepdims=True), fold_ref[...],
                  preferred_element_type=F32) * 0.25
    return g_t * xc * lax.rsqrt(var + 1e-5) + be_t


def _combine_body(s0_ref, s1_ref, z_ref, p_ref, fold_ref, degb_ref,
                  w_ref, wz_ref, y_ref, zo_ref, *, q, layer, relu):
    hb = _combine_common(s0_ref, s1_ref, z_ref, p_ref, fold_ref, degb_ref,
                         q, layer, relu)
    y = jnp.dot(hb, w_ref[...], preferred_element_type=F32)
    col = lax.broadcasted_iota(I32, (q, 4 * _W), 1)
    y_ref[...] = jnp.where(col % _W == _W - 1, 1.0, y)
    zo_ref[...] = jnp.dot(hb, wz_ref[...], preferred_element_type=F32)


def _tc_combine(s0, s1, z, p, fold, degb, w, wz, layer, relu):
    q = z.shape[0]
    return pl.pallas_call(
        functools.partial(_combine_body, q=q, layer=layer, relu=relu),
        out_shape=[jax.ShapeDtypeStruct((q, 128), F32),
                   jax.ShapeDtypeStruct((q, 128), F32)],
    )(s0, s1, z, p, fold, degb, w, wz)


def _final_body(s0_ref, s1_ref, z_ref, p_ref, fold_ref, degb_ref,
                o_ref, *, q, q2):
    hb = _combine_common(s0_ref, s1_ref, z_ref, p_ref, fold_ref, degb_ref,
                         q, layer=2, relu=False)
    o_ref[...] = jnp.concatenate(
        [hb, jnp.full((q2 - q, 128), -1e30, F32)], axis=0)


def _tc_final(s0, s1, z, p, fold, degb, q2):
    q = z.shape[0]
    return pl.pallas_call(
        functools.partial(_final_body, q=q, q2=q2),
        out_shape=jax.ShapeDtypeStruct((q2, 128), F32),
    )(s0, s1, z, p, fold, degb)


def _head_body(t_ref, w1_ref, b1_ref, w2_ref, b2_ref, o_ref, *, h):
    t = t_ref[...].reshape(_NW, 32, 128)
    p = jnp.max(t, axis=0)
    p = jnp.where(p < -1e29, 0.0, p)
    p = p.reshape(128, _W)[:, :h]
    zz = jnp.dot(p, w1_ref[...], preferred_element_type=F32) \
        + b1_ref[...][None, :]
    zz = jnp.maximum(zz, 0.0)
    o = jnp.dot(zz, w2_ref[...], preferred_element_type=F32) \
        + b2_ref[...][None, :]
    o_ref[...] = 1.0 / (1.0 + jnp.exp(-2.0 * o))


def _tc_head(t, w1, b1, w2, b2):
    return pl.pallas_call(
        functools.partial(_head_body, h=w1.shape[0]),
        out_shape=jax.ShapeDtypeStruct((128, 1), F32),
    )(t, w1, b1, w2, b2)


def _make_segsum(n_nodes, n_ch):
    npad = ((n_nodes + 1 + _NSUB * 8 - 1) // (_NSUB * 8)) * (_NSUB * 8)
    rpt = npad // _NSUB
    mesh = plsc.VectorSubcoreMesh(core_axis_name="c", subcore_axis_name="s")
    K = 5
    ng = n_ch // K
    assert n_ch % (2 * K) == 0 and ng >= 4

    @functools.partial(
        pl.kernel, mesh=mesh,
        compiler_params=pltpu.CompilerParams(use_tc_tiling_on_sc=False),
        out_type=[jax.ShapeDtypeStruct((npad, _W), F32),
                  jax.ShapeDtypeStruct((npad, _W), F32)],
        scratch_types=[
            pltpu.VMEM_SHARED((npad, _W), F32),
            pltpu.VMEM((n_ch, _CB), I32),
            pltpu.VMEM((n_ch, _CB), I32),
            pltpu.VMEM((K, _CB, _W), F32),
            pltpu.VMEM((K, _CB, _W), F32),
            pltpu.SemaphoreType.DMA,
            pltpu.SemaphoreType.DMA,
            pltpu.SemaphoreType.DMA,
            pltpu.SemaphoreType.DMA,
        ],
    )
    def segsum(y_hbm, src_hbm, dst_hbm, zeros_hbm, out0, out1,
               acc, src_v, dst_v, rows0, rows1,
               sem_g0, sem_g1, sem_s0, sem_s1):
        c = lax.axis_index("c")
        s = lax.axis_index("s")
        wid = c * _NSUB + s
        rs = pl.ds(s * rpt, rpt)
        rows = (rows0, rows1)
        semg = (sem_g0, sem_g1)
        sems = (sem_s0, sem_s1)
        pltpu.sync_copy(src_hbm.at[wid], src_v)

        def fire_g(g, b):
            for k in range(K):
                pltpu.async_copy(y_hbm.at[src_v.at[g * K + k]],
                                 rows[b].at[k], semg[b])

        def drain_g(g, b):
            for k in range(K):
                pltpu.make_async_copy(y_hbm.at[src_v.at[g * K + k]],
                                      rows[b].at[k], semg[b]).wait()

        def fire_s(g, b):
            for k in range(K):
                pltpu.async_copy(rows[b].at[k],
                                 acc.at[dst_v.at[g * K + k]], sems[b],
                                 add=True)

        def drain_s(g, b):
            for k in range(K):
                pltpu.make_async_copy(rows[b].at[k],
                                      acc.at[dst_v.at[g * K + k]],
                                      sems[b]).wait()

        fire_g(0, 0)
        pltpu.sync_copy(dst_hbm.at[wid], dst_v)
        pltpu.sync_copy(zeros_hbm, acc.at[rs])
        plsc.subcore_barrier()
        drain_g(0, 0)
        fire_s(0, 0)
        fire_g(1, 1)

        def pair(t, carry):
            g1 = 2 * t + 1
            drain_g(g1, 1)
            fire_s(g1, 1)
            drain_s(g1 - 1, 0)
            fire_g(g1 + 1, 0)
            g2 = 2 * t + 2
            drain_g(g2, 0)
            fire_s(g2, 0)
            drain_s(g2 - 1, 1)
            fire_g(g2 + 1, 1)
            return carry
        lax.fori_loop(0, (ng - 2) // 2, pair, 0)

        g_last = ng - 1
        drain_g(g_last, 1)
        fire_s(g_last, 1)
        drain_s(g_last - 1, 0)
        drain_s(g_last, 1)

        plsc.subcore_barrier()

        @pl.when(c == 0)
        def _():
            pltpu.sync_copy(acc.at[rs], out0.at[rs])

        @pl.when(c == 1)
        def _():
            pltpu.sync_copy(acc.at[rs], out1.at[rs])

    return segsum, npad


def _make_segmax(n_per_tile, n_graphs):
    mesh = plsc.VectorSubcoreMesh(core_axis_name="c", subcore_axis_name="s")

    @functools.partial(
        pl.kernel, mesh=mesh,
        compiler_params=pltpu.CompilerParams(use_tc_tiling_on_sc=False),
        out_type=jax.ShapeDtypeStruct((_NW, n_graphs, _W), F32),
        scratch_types=[
            pltpu.VMEM((n_per_tile, _W), F32),
            pltpu.VMEM((n_per_tile,), I32),
            pltpu.VMEM((n_graphs, _W), F32),
        ],
    )
    def segmax(h_hbm, batch_hbm, neg_hbm, out, h_v, b_v, acc_v):
        c = lax.axis_index("c")
        s = lax.axis_index("s")
        wid = c * _NSUB + s
        n0 = wid * n_per_tile
        pltpu.sync_copy(h_hbm.at[pl.ds(n0, n_per_tile)], h_v)
        pltpu.sync_copy(batch_hbm.at[pl.ds(n0, n_per_tile)], b_v)
        pltpu.sync_copy(neg_hbm, acc_v)

        def body(j, carry):
            n0 = j * 16
            gvec = b_v[pl.ds(n0, 16)]
            for k in range(16):
                g = gvec[k]
                nk = n0 + k
                lo = acc_v[g, pl.ds(0, 16)]
                hi = acc_v[g, pl.ds(16, 16)]
                acc_v[g, pl.ds(0, 16)] = jnp.maximum(lo, h_v[nk, pl.ds(0, 16)])
                acc_v[g, pl.ds(16, 16)] = jnp.maximum(hi, h_v[nk, pl.ds(16, 16)])
            return carry
        lax.fori_loop(0, n_per_tile // 16, body, 0)

        pltpu.sync_copy(acc_v, out.at[wid])

    return segmax


def kernel(x, edge_index, edge_attr, batch,
           Wl1, Wr1, b1, Wl2, Wr2, b2, Wl3, Wr3, b3,
           g1, be1, g2, be2, g3, be3, lin1_W, lin1_b, lin2_W, lin2_b):
    n, f_in = x.shape
    e = edge_index.shape[1]
    h = Wl1.shape[1]
    g_graphs = 128

    q = n // 4

    n_ch = math.ceil(e / (_NW * _CB))
    e_pad = _NW * n_ch * _CB
    segsum, npad = _make_segsum(n, n_ch)

    pad_i = np.arange(e_pad - e)
    src_pad = jnp.asarray(4 * (pad_i % q), dtype=I32)
    dst_pad = jnp.asarray(n + pad_i % (npad - n), dtype=I32)
    e2d = edge_index.reshape(2 * e // 128, 128)
    zeros_t = jnp.asarray(np.zeros((npad // _NSUB, _W), np.float32))

    npt = math.ceil(n / (_NW * 8)) * 8
    n2 = _NW * npt
    batch_packed = batch.reshape(4, q).T.reshape(n)
    batch_p = jnp.concatenate([batch_packed, jnp.zeros((n2 - n,), I32)])
    neg = jnp.asarray(np.full((g_graphs, _W), -1e30, np.float32))
    segmax = _make_segmax(npt, g_graphs)

    def pad32(wl):
        w = jnp.zeros((_W, _W), F32)
        return w.at[:h, :h].set(wl)

    def blockdiag4(w32):
        z = jnp.zeros((_W, _W), F32)
        return jnp.block([[w32 if i == j else z for j in range(4)]
                          for i in range(4)])

    w1c = jnp.zeros((f_in, 2 * _W), F32)
    w1c = w1c.at[:, :h].set(Wl1).at[:, _W:_W + h].set(Wr1)
    w2y, w2z = blockdiag4(pad32(Wl2)), blockdiag4(pad32(Wr2))
    w3y, w3z = blockdiag4(pad32(Wl3)), blockdiag4(pad32(Wr3))

    bn_p = jnp.stack([b1, g1, be1, b2, g2, be2, b3, g3, be3])

    lane = np.arange(128)
    fold = jnp.asarray(
        (lane[:, None] % _W == lane[None, :] % _W).astype(np.float32))
    degb = jnp.asarray(
        (lane[:, None] == _W * (lane[None, :] // _W) + _W - 1)
        .astype(np.float32))

    def sc_view(ypk):
        return ypk.reshape(n, _W)

    def tc_view(part):
        return part.reshape(npad // 4, 128)

    y1, z1, t2d = _tc_proj(x, w1c, e2d)
    t_flat = t2d.reshape(2 * e)
    src = jnp.concatenate([t_flat[:e], src_pad])
    dst = jnp.concatenate([t_flat[e:], dst_pad])
    src_r = src.reshape(_NW, n_ch, _CB)
    dst_r = dst.reshape(_NW, n_ch, _CB)
    p0, p1 = segsum(sc_view(y1), src_r, dst_r, zeros_t)
    y2, z2 = _tc_combine(tc_view(p0), tc_view(p1), z1, bn_p,
                         fold, degb, w2y, w2z, layer=0, relu=True)
    p0, p1 = segsum(sc_view(y2), src_r, dst_r, zeros_t)
    y3, z3 = _tc_combine(tc_view(p0), tc_view(p1), z2, bn_p,
                         fold, degb, w3y, w3z, layer=1, relu=True)
    p0, p1 = segsum(sc_view(y3), src_r, dst_r, zeros_t)
    h3 = _tc_final(tc_view(p0), tc_view(p1), z3, bn_p,
                   fold, degb, n2 // 4)
    t = segmax(h3.reshape(n2, _W), batch_p, neg)
    return _tc_head(t.reshape(_NW * g_graphs // 4, 128),
                    lin1_W, lin1_b, lin2_W, lin2_b)
